# baseline (speedup 1.0000x reference)
import functools

import jax
import jax.numpy as jnp
from jax import lax
from jax.experimental import pallas as pl
from jax.experimental.pallas import tpu as pltpu
from jax.experimental.pallas import tpu_sc as plsc

E = 64
C = 64
D = 768
H = 768
O = 768
B = 2048
S = E * C


def _gating_body(x_ref, wg_ref, bg_ref, dest_ref, scat_ref, gate_ref,
                 counts_ref, aux_ref):
    x = x_ref[...]
    wg = wg_ref[...]
    logits = lax.dot_general(x, wg, (((1,), (1,)), ((), ())),
                             preferred_element_type=jnp.float32) + bg_ref[...]
    m = jnp.max(logits, axis=1, keepdims=True)
    p = jnp.exp(logits - m)
    probs = p / jnp.sum(p, axis=1, keepdims=True)
    amax = jnp.max(probs, axis=1, keepdims=True)
    iot = lax.broadcasted_iota(jnp.int32, (B, E), 1)
    eidx = jnp.min(jnp.where(probs >= amax, iot, E), axis=1)
    gate = amax[:, 0]
    onehot = (iot == eidx[:, None]).astype(jnp.float32)

    G, g = B // 8, 8
    oh3 = jnp.reshape(onehot, (G, g, E))
    gs = jnp.sum(oh3, axis=1)
    rg = lax.broadcasted_iota(jnp.int32, (G, G), 0)
    cg = lax.broadcasted_iota(jnp.int32, (G, G), 1)
    trilg = (cg < rg).astype(jnp.float32)
    gp = lax.dot_general(trilg, gs, (((1,), (0,)), ((), ())),
                         preferred_element_type=jnp.float32)
    parts = [jnp.zeros((G, 1, E), jnp.float32)]
    acc = oh3[:, 0:1, :]
    for i in range(1, g):
        parts.append(acc)
        acc = acc + oh3[:, i:i + 1, :]
    wp = jnp.concatenate(parts, axis=1)
    pref3 = gp[:, None, :] + wp
    rank = jnp.reshape(jnp.sum(pref3 * oh3, axis=2), (B,)).astype(jnp.int32)

    counts = jnp.sum(onehot, axis=0)

    importance = jnp.sum(probs, axis=0)
    mean_imp = jnp.sum(importance) / E
    imp_loss = jnp.sum((importance - mean_imp) ** 2) / ((E - 1) * E * E)
    usage = counts / B
    rw = jnp.sum(probs * onehot, axis=0) / B
    lb = E * jnp.sum(usage * rw)

    valid = rank < C
    dest_v = eidx * C + rank
    minc = jnp.min(counts)
    iot_e = lax.broadcasted_iota(jnp.int32, (E,), 0)
    estar = jnp.min(jnp.where(counts <= minc, iot_e, E))
    empty_slot = estar * C + minc.astype(jnp.int32)
    dest_ref[0, :] = jnp.where(valid, dest_v, empty_slot)
    scat_ref[0, :] = jnp.where(valid, dest_v, S)
    gate_ref[0, :] = gate
    counts_ref[0, :] = counts
    aux_ref[...] = jnp.reshape(imp_loss + lb, (1, 1))


def _gating(x, Wg, bg):
    return pl.pallas_call(
        _gating_body,
        out_shape=(
            jax.ShapeDtypeStruct((1, B), jnp.int32),
            jax.ShapeDtypeStruct((1, B), jnp.int32),
            jax.ShapeDtypeStruct((1, B), jnp.float32),
            jax.ShapeDtypeStruct((1, E), jnp.float32),
            jax.ShapeDtypeStruct((1, 1), jnp.float32),
        ),
    )(x, Wg, bg.reshape(1, E))


EPS = 2


def _ffn_body(xin_ref, w1_ref, b1_ref, w2_ref, b2_ref, ws_ref, y_ref):
    for u in range(EPS):
        xb = xin_ref[pl.ds(u * C, C), :]
        h = lax.dot_general(xb, w1_ref[u], (((1,), (1,)), ((), ())),
                            preferred_element_type=jnp.float32) + b1_ref[u]
        h = 0.5 * h * (1.0 + lax.erf(h * 0.7071067811865476))
        y = lax.dot_general(h, w2_ref[u], (((1,), (1,)), ((), ())),
                            preferred_element_type=jnp.float32) + b2_ref[u]
        ws = jnp.reshape(ws_ref[u, 0, :], (C, 1))
        y_ref[pl.ds(u * C, C), :] = y * ws


def _ffn(xin, W1, b1, W2, b2, wslot):
    return pl.pallas_call(
        _ffn_body,
        grid=(E // EPS,),
        in_specs=[
            pl.BlockSpec((EPS * C, D), lambda i: (i, 0)),
            pl.BlockSpec((EPS, H, D), lambda i: (i, 0, 0)),
            pl.BlockSpec((EPS, 1, H), lambda i: (i, 0, 0)),
            pl.BlockSpec((EPS, O, H), lambda i: (i, 0, 0)),
            pl.BlockSpec((EPS, 1, O), lambda i: (i, 0, 0)),
            pl.BlockSpec((EPS, 1, C), lambda i: (i, 0, 0)),
        ],
        out_specs=pl.BlockSpec((EPS * C, O), lambda i: (i, 0)),
        out_shape=jax.ShapeDtypeStruct((S, O), jnp.float32),
        compiler_params=pltpu.CompilerParams(
            dimension_semantics=("arbitrary",)),
    )(xin, W1, b1.reshape(E, 1, H), W2, b2.reshape(E, 1, O),
      wslot.reshape(E, 1, C))


def _row_gather(table, idx, n_rows, d):
    mesh = plsc.VectorSubcoreMesh(core_axis_name="c", subcore_axis_name="s")
    nc = mesh.num_cores
    nw = nc * mesh.num_subcores
    per = n_rows // nw

    @functools.partial(
        pl.kernel,
        out_type=jax.ShapeDtypeStruct((n_rows, d), jnp.float32),
        mesh=mesh,
        scratch_types=[
            pltpu.VMEM((per,), jnp.int32),
            pltpu.VMEM((per, d), jnp.float32),
            pltpu.SemaphoreType.DMA,
        ],
    )
    def k(table_hbm, idx_hbm, out_hbm, idx_v, rows_v, sem):
        wid = lax.axis_index("s") * nc + lax.axis_index("c")
        base = wid * per
        pltpu.sync_copy(idx_hbm.at[pl.ds(base, per)], idx_v)
        pltpu.async_copy(table_hbm.at[idx_v], rows_v, sem).wait()
        pltpu.sync_copy(rows_v, out_hbm.at[pl.ds(base, per)])

    return k(table, idx)


def kernel(x, Wg, bg, W1, b1, W2, b2):
    dest2, scat2, gate2, counts2, aux = _gating(x, Wg, bg)
    src = (jnp.arange(S, dtype=jnp.int32) % B).at[scat2[0]].set(
        jnp.arange(B, dtype=jnp.int32), mode="drop")

    occ = (lax.broadcasted_iota(jnp.int32, (E, C), 1)
           < counts2[0].astype(jnp.int32)[:, None]).reshape(S)
    wslot = jnp.where(occ, jnp.take(gate2[0], src), 0.0)

    xin = _row_gather(x, src, S, D)
    y = _ffn(xin, W1, b1, W2, b2, wslot)
    out = _row_gather(y, dest2[0], B, O)
    return out, aux[0, 0]


# speedup vs baseline: 1.1489x; 1.1489x over previous
import functools

import jax
import jax.numpy as jnp
from jax import lax
from jax.experimental import pallas as pl
from jax.experimental.pallas import tpu as pltpu
from jax.experimental.pallas import tpu_sc as plsc

E = 64
C = 64
D = 768
H = 768
O = 768
B = 2048
S = E * C


def _gating_body(x_ref, wg_ref, bg_ref, dest_ref, scat_ref, gate_ref,
                 aux_ref):
    x = x_ref[...]
    wg = wg_ref[...]
    logits = lax.dot_general(x, wg, (((1,), (1,)), ((), ())),
                             preferred_element_type=jnp.float32) + bg_ref[...]
    m = jnp.max(logits, axis=1, keepdims=True)
    p = jnp.exp(logits - m)
    probs = p / jnp.sum(p, axis=1, keepdims=True)
    amax = jnp.max(probs, axis=1, keepdims=True)
    iot = lax.broadcasted_iota(jnp.int32, (B, E), 1)
    eidx = jnp.min(jnp.where(probs >= amax, iot, E), axis=1)
    gate = amax[:, 0]
    onehot = (iot == eidx[:, None]).astype(jnp.float32)

    G, g = B // 8, 8
    oh3 = jnp.reshape(onehot, (G, g, E))
    gs = jnp.sum(oh3, axis=1)
    rg = lax.broadcasted_iota(jnp.int32, (G, G), 0)
    cg = lax.broadcasted_iota(jnp.int32, (G, G), 1)
    trilg = (cg < rg).astype(jnp.float32)
    gp = lax.dot_general(trilg, gs, (((1,), (0,)), ((), ())),
                         preferred_element_type=jnp.float32)
    parts = [jnp.zeros((G, 1, E), jnp.float32)]
    acc = oh3[:, 0:1, :]
    for i in range(1, g):
        parts.append(acc)
        acc = acc + oh3[:, i:i + 1, :]
    wp = jnp.concatenate(parts, axis=1)
    pref3 = gp[:, None, :] + wp
    rank = jnp.reshape(jnp.sum(pref3 * oh3, axis=2), (B,)).astype(jnp.int32)

    counts = jnp.sum(onehot, axis=0)

    importance = jnp.sum(probs, axis=0)
    mean_imp = jnp.sum(importance) / E
    imp_loss = jnp.sum((importance - mean_imp) ** 2) / ((E - 1) * E * E)
    usage = counts / B
    rw = jnp.sum(probs * onehot, axis=0) / B
    lb = E * jnp.sum(usage * rw)

    valid = rank < C
    dest_v = eidx * C + rank
    minc = jnp.min(counts)
    iot_e = lax.broadcasted_iota(jnp.int32, (E,), 0)
    estar = jnp.min(jnp.where(counts <= minc, iot_e, E))
    empty_slot = estar * C + minc.astype(jnp.int32)
    dest_ref[0, :] = jnp.where(valid, dest_v, empty_slot)
    scat_ref[0, :] = jnp.where(valid, dest_v, S)
    gate_ref[0, :] = gate
    aux_ref[...] = jnp.reshape(imp_loss + lb, (1, 1))


def _gating(x, Wg, bg):
    return pl.pallas_call(
        _gating_body,
        out_shape=(
            jax.ShapeDtypeStruct((1, B), jnp.int32),
            jax.ShapeDtypeStruct((1, B), jnp.int32),
            jax.ShapeDtypeStruct((1, B), jnp.float32),
            jax.ShapeDtypeStruct((1, 1), jnp.float32),
        ),
    )(x, Wg, bg.reshape(1, E))


EPS = 2


def _ffn_body(xin_ref, w1_ref, b1_ref, w2_ref, b2_ref, ws_ref, y_ref):
    for u in range(EPS):
        xb = xin_ref[pl.ds(u * C, C), :]
        h = lax.dot_general(xb, w1_ref[u], (((1,), (1,)), ((), ())),
                            preferred_element_type=jnp.float32) + b1_ref[u]
        h = 0.5 * h * (1.0 + lax.erf(h * 0.7071067811865476))
        y = lax.dot_general(h, w2_ref[u], (((1,), (1,)), ((), ())),
                            preferred_element_type=jnp.float32) + b2_ref[u]
        ws = jnp.reshape(ws_ref[u, 0, :], (C, 1))
        y_ref[pl.ds(u * C, C), :] = y * ws


def _ffn(xin, W1, b1, W2, b2, wslot):
    return pl.pallas_call(
        _ffn_body,
        grid=(E // EPS,),
        in_specs=[
            pl.BlockSpec((EPS * C, D), lambda i: (i, 0)),
            pl.BlockSpec((EPS, H, D), lambda i: (i, 0, 0)),
            pl.BlockSpec((EPS, 1, H), lambda i: (i, 0, 0)),
            pl.BlockSpec((EPS, O, H), lambda i: (i, 0, 0)),
            pl.BlockSpec((EPS, 1, O), lambda i: (i, 0, 0)),
            pl.BlockSpec((EPS, 1, C), lambda i: (i, 0, 0)),
        ],
        out_specs=pl.BlockSpec((EPS * C, O), lambda i: (i, 0)),
        out_shape=jax.ShapeDtypeStruct((S, O), jnp.float32),
        compiler_params=pltpu.CompilerParams(
            dimension_semantics=("arbitrary",)),
    )(xin, W1, b1.reshape(E, 1, H), W2, b2.reshape(E, 1, O),
      wslot.reshape(E, 1, C))


def _row_gather(table, idx, n_rows, d):
    mesh = plsc.VectorSubcoreMesh(core_axis_name="c", subcore_axis_name="s")
    nc = mesh.num_cores
    nw = nc * mesh.num_subcores
    per = n_rows // nw

    @functools.partial(
        pl.kernel,
        out_type=jax.ShapeDtypeStruct((n_rows, d), jnp.float32),
        mesh=mesh,
        scratch_types=[
            pltpu.VMEM((per,), jnp.int32),
            pltpu.VMEM((per, d), jnp.float32),
            pltpu.SemaphoreType.DMA,
        ],
    )
    def k(table_hbm, idx_hbm, out_hbm, idx_v, rows_v, sem):
        wid = lax.axis_index("s") * nc + lax.axis_index("c")
        base = wid * per
        pltpu.sync_copy(idx_hbm.at[pl.ds(base, per)], idx_v)
        pltpu.async_copy(table_hbm.at[idx_v], rows_v, sem).wait()
        pltpu.sync_copy(rows_v, out_hbm.at[pl.ds(base, per)])

    return k(table, idx)


def kernel(x, Wg, bg, W1, b1, W2, b2):
    dest2, scat2, gate2, aux = _gating(x, Wg, bg)
    scat_idx = scat2[0]
    src = (jnp.arange(S, dtype=jnp.int32) % B).at[scat_idx].set(
        jnp.arange(B, dtype=jnp.int32), mode="drop")
    wslot = jnp.zeros((S,), jnp.float32).at[scat_idx].set(
        gate2[0], mode="drop")

    xin = _row_gather(x, src, S, D)
    y = _ffn(xin, W1, b1, W2, b2, wslot)
    out = _row_gather(y, dest2[0], B, O)
    return out, aux[0, 0]
